# Initial kernel scaffold; baseline (speedup 1.0000x reference)
import jax
import jax.numpy as jnp
from jax.experimental import pallas as pl


def kernel(sentence, context, W, neg_samples):
    raise NotImplementedError("write your pallas kernel here")


# speedup vs baseline: 3.4092x; 3.4092x over previous
import functools

import jax
import jax.numpy as jnp
from jax import lax
from jax.experimental import pallas as pl
from jax.experimental.pallas import tpu as pltpu
from jax.experimental.pallas import tpu_sc as plsc

B = 16384
D = 128
V = 1000
K = 5
L = 16
NC = 2
NS = 16
NW = NC * NS
S = B // NW
C = 64
NCHUNK = S // C
ROWS_N = C * K
NIDX_MINOR = 40
NIDX_ROWS = ROWS_N // NIDX_MINOR
RPS = K + 1
PART_ROWS = C * RPS * L // D
OUT_ROWS = B * RPS * L // D


def _sc_body(sent_hbm, neg_hbm, ctx_hbm, w_hbm, out_hbm,
             sidx_v, nidx_v, ctx_v, srows_v, nrows_v, part_v, sem):
    wid = lax.axis_index("s") * NC + lax.axis_index("c")

    for chunk in range(NCHUNK):
        base = pl.multiple_of(wid * S + chunk * C, C)
        pltpu.sync_copy(sent_hbm.at[pl.ds(base, C)], sidx_v)
        off_n = pl.multiple_of(base * K // NIDX_MINOR, 8)
        pltpu.sync_copy(neg_hbm.at[pl.ds(off_n, NIDX_ROWS)], nidx_v)
        pltpu.sync_copy(ctx_hbm.at[pl.ds(base, C)], ctx_v)
        copies = [pltpu.async_copy(w_hbm.at[sidx_v], srows_v, sem)]
        for t in range(NIDX_ROWS):
            copies.append(pltpu.async_copy(
                w_hbm.at[nidx_v.at[t]],
                nrows_v.at[pl.ds(t * NIDX_MINOR, NIDX_MINOR)], sem))
        for cp in copies:
            cp.wait()

        def quad_body(i4, _):
            pr_base = 3 * i4
            for u in range(4):
                i = 4 * i4 + u
                c = [ctx_v[i, pl.ds(L * j, L)] for j in range(D // L)]
                acc = c[0] * srows_v[i, pl.ds(0, L)]
                for j in range(1, D // L):
                    acc = acc + c[j] * srows_v[i, pl.ds(L * j, L)]
                rr = RPS * u
                part_v[pr_base + rr // 8, pl.ds(L * (rr % 8), L)] = acc
                for k in range(K):
                    r = K * i + k
                    accn = c[0] * nrows_v[r, pl.ds(0, L)]
                    for j in range(1, D // L):
                        accn = accn + c[j] * nrows_v[r, pl.ds(L * j, L)]
                    rr = RPS * u + 1 + k
                    part_v[pr_base + rr // 8, pl.ds(L * (rr % 8), L)] = -accn
            return 0

        lax.fori_loop(0, C // 4, quad_body, 0)
        off_o = pl.multiple_of(base * RPS * L // D, 8)
        pltpu.sync_copy(part_v, out_hbm.at[pl.ds(off_o, PART_ROWS)])


@functools.partial(
    pl.kernel,
    out_type=pltpu.HBM((OUT_ROWS, D), jnp.float32),
    mesh=plsc.VectorSubcoreMesh(core_axis_name="c", subcore_axis_name="s"),
    scratch_types=[
        pltpu.VMEM((C,), jnp.int32),
        pltpu.VMEM((NIDX_ROWS, NIDX_MINOR), jnp.int32),
        pltpu.VMEM((C, D), jnp.float32),
        pltpu.VMEM((C, D), jnp.float32),
        pltpu.VMEM((ROWS_N, D), jnp.float32),
        pltpu.VMEM((PART_ROWS, D), jnp.float32),
        pltpu.SemaphoreType.DMA,
    ],
)
def _sc_dots(sent_hbm, neg_hbm, ctx_hbm, w_hbm, out_hbm,
             sidx_v, nidx_v, ctx_v, srows_v, nrows_v, part_v, sem):
    _sc_body(sent_hbm, neg_hbm, ctx_hbm, w_hbm, out_hbm,
             sidx_v, nidx_v, ctx_v, srows_v, nrows_v, part_v, sem)


_TC_ROWS = 1024


def _tc_body(p_ref, o_ref):
    x = p_ref[...]
    col = lax.broadcasted_iota(jnp.int32, (D, 8), 0)
    grp = lax.broadcasted_iota(jnp.int32, (D, 8), 1)
    m = (col // L == grp).astype(jnp.float32)
    d = jnp.dot(x, m, preferred_element_type=jnp.float32)
    ls = jnp.minimum(d, 0.0) - jnp.log1p(jnp.exp(-jnp.abs(d)))

    @pl.when(pl.program_id(0) == 0)
    def _():
        o_ref[0, 0] = 0.0

    o_ref[0, 0] -= jnp.sum(ls)


def _tc_reduce(part):
    grid = OUT_ROWS // _TC_ROWS
    out = pl.pallas_call(
        _tc_body,
        grid=(grid,),
        in_specs=[pl.BlockSpec((_TC_ROWS, D), lambda i: (i, 0))],
        out_specs=pl.BlockSpec(memory_space=pltpu.SMEM,
                               block_shape=(1, 1), index_map=lambda i: (0, 0)),
        out_shape=jax.ShapeDtypeStruct((1, 1), jnp.float32),
    )(part)
    return out[0, 0]


def kernel(sentence, context, W, neg_samples):
    neg2d = neg_samples.reshape(B * K // NIDX_MINOR, NIDX_MINOR)
    part = _sc_dots(sentence, neg2d, context, W)
    return _tc_reduce(part)
